# baseline (speedup 1.0000x reference)
import jax
import jax.numpy as jnp
from jax.experimental import pallas as pl
from jax.experimental.pallas import tpu as pltpu

_B = 1000


def _heads_kernel(x_ref, wc_hbm, bc_ref, wb_hbm, bb_ref, s_ref, d_ref,
                  wc_v, wb_v, sem):
    i = pl.program_id(0)

    @pl.when(i == 0)
    def _():
        pltpu.make_async_copy(wc_hbm, wc_v, sem.at[0]).start()
        pltpu.make_async_copy(wb_hbm, wb_v, sem.at[1]).start()
        pltpu.make_async_copy(wc_hbm, wc_v, sem.at[0]).wait()
        pltpu.make_async_copy(wb_hbm, wb_v, sem.at[1]).wait()

    x = x_ref[...].astype(jnp.bfloat16)
    s_ref[...] = (
        jnp.dot(x, wc_v[...], preferred_element_type=jnp.float32) + bc_ref[...]
    )
    d_ref[...] = (
        jnp.dot(x, wb_v[...], preferred_element_type=jnp.float32) + bb_ref[...]
    )


def kernel(x, W_cls, b_cls, W_bbox, b_bbox):
    if x.ndim > 2:
        x = x.reshape(x.shape[0], -1)
    N, D = x.shape
    Kc = W_cls.shape[1]
    Kb = W_bbox.shape[1]
    bc2 = b_cls.reshape(1, Kc)
    bb2 = b_bbox.reshape(1, Kb)
    Wc16 = W_cls.astype(jnp.bfloat16)
    Wb16 = W_bbox.astype(jnp.bfloat16)
    grid = (N // _B,)
    scores, deltas = pl.pallas_call(
        _heads_kernel,
        grid=grid,
        in_specs=[
            pl.BlockSpec((_B, D), lambda i: (i, 0)),
            pl.BlockSpec(memory_space=pl.ANY),
            pl.BlockSpec((1, Kc), lambda i: (0, 0)),
            pl.BlockSpec(memory_space=pl.ANY),
            pl.BlockSpec((1, Kb), lambda i: (0, 0)),
        ],
        out_specs=[
            pl.BlockSpec((_B, Kc), lambda i: (i, 0)),
            pl.BlockSpec((_B, Kb), lambda i: (i, 0)),
        ],
        out_shape=[
            jax.ShapeDtypeStruct((N, Kc), jnp.float32),
            jax.ShapeDtypeStruct((N, Kb), jnp.float32),
        ],
        scratch_shapes=[
            pltpu.VMEM((D, Kc), jnp.bfloat16),
            pltpu.VMEM((D, Kb), jnp.bfloat16),
            pltpu.SemaphoreType.DMA((2,)),
        ],
        compiler_params=pltpu.CompilerParams(
            dimension_semantics=("arbitrary",),
        ),
    )(x, Wc16, bc2, Wb16, bb2)
    return (scores, deltas)
